# Initial kernel scaffold; baseline (speedup 1.0000x reference)
import jax
import jax.numpy as jnp
from jax.experimental import pallas as pl


def kernel(sent_ids, edge_index, edge_type, node_ids, emb, basis, comp, root, bias, WQ, WK, WV, attW, attb, fcW, fcb):
    raise NotImplementedError("write your pallas kernel here")


# speedup vs baseline: 143.8342x; 143.8342x over previous
import functools

import jax
import jax.numpy as jnp
from jax import lax
from jax.experimental import pallas as pl
from jax.experimental.pallas import tpu as pltpu
from jax.experimental.pallas import tpu_sc as plsc

B = 4
N_NODES = 10000
E = 160000
L = 50
D = 128
H = 128
HEADS = 3
R = 5
NB = 4
VOCAB = 100000

NC = 2
NS = 16
NW = NC * NS
EPT = 5120
PE = NW * EPT
GROUPS = EPT // 16

SROWS = 304
XBASE = 1 + R * L
CAP = 512
NCH = CAP // 32
LP = 64


def _sc_body(ei_hbm, et_hbm, nids_hbm, sent_hbm, emb_hbm, s_out, deg_out,
             marker, nids, srcb, dstb, etb, sentb, idx_s, idx_o, idx_m,
             rows, ones_r, zrows, z16, sg, degg, sem):
    cc = lax.axis_index("c")
    ss = lax.axis_index("s")
    wid = cc * NS + ss

    zero16f = jnp.zeros((16,), jnp.float32)
    one16f = jnp.ones((16,), jnp.float32)
    zero16i = jnp.zeros((16,), jnp.int32)
    neg16i = jnp.full((16,), -1, jnp.int32)
    lane = lax.iota(jnp.int32, 16)

    def _init_const(i, _):
        for k in range(8):
            zrows[i, pl.ds(k * 16, 16)] = zero16f
        z16[i, pl.ds(0, 16)] = zero16f
        ones_r[i, pl.ds(0, 16)] = one16f
        ones_r[i + 16, pl.ds(0, 16)] = one16f
        return 0
    lax.fori_loop(0, 16, _init_const, 0)

    idx = 0
    for g in range(B):
        for ch in range(SROWS // 16):
            @pl.when(ss == (idx % NS))
            def _():
                pltpu.sync_copy(zrows, sg.at[g, pl.ds(ch * 16, 16)])
            idx += 1
    for g in range(B):
        for ch in range(LP // 16):
            @pl.when(ss == ((g * (LP // 16) + ch) % NS))
            def _():
                pltpu.sync_copy(z16, degg.at[g, pl.ds(ch * 16, 16)])
    plsc.subcore_barrier()

    ebase = wid * EPT
    for g in range(B):
        def _memset(i, _):
            marker[pl.ds(i * 16, 16)] = neg16i
            return 0
        lax.fori_loop(0, N_NODES // 16, _memset, 0)
        pltpu.sync_copy(nids_hbm.at[g], nids)
        pltpu.sync_copy(sent_hbm.at[g], sentb)
        for k in range(LP // 16):
            sv = sentb[pl.ds(k * 16, 16)]
            lv = lane + k * 16
            plsc.store_scatter(marker, [sv], lv, mask=lv < L)

        pltpu.sync_copy(ei_hbm.at[g, 0, pl.ds(ebase, EPT)], srcb)
        pltpu.sync_copy(ei_hbm.at[g, 1, pl.ds(ebase, EPT)], dstb)
        pltpu.sync_copy(et_hbm.at[g, pl.ds(ebase, EPT)], etb)

        def _prefill(ch, _):
            for k in range(2):
                idx_s[ch, pl.ds(k * 16, 16)] = zero16i
                idx_o[ch, pl.ds(k * 16, 16)] = zero16i
                idx_m[ch, pl.ds(k * 16, 16)] = zero16i
            return 0
        lax.fori_loop(0, NCH, _prefill, 0)

        def _scan(j, off):
            d = dstb[pl.ds(j * 16, 16)]
            s = srcb[pl.ds(j * 16, 16)]
            t = etb[pl.ds(j * 16, 16)]
            m = plsc.load_gather(marker, [d])
            valid = (m >= 0) & (ebase + j * 16 + lane < E)
            nid = plsc.load_gather(nids, [s])
            oidx = 1 + t * L + m
            midx = 1 + m
            cs = plsc.cumsum(valid.astype(jnp.int32))
            pos = jnp.minimum(off + cs - 1, CAP - 1)
            hi = pos >> 5
            lo = pos & 31
            plsc.store_scatter(idx_s, [hi, lo], nid, mask=valid)
            plsc.store_scatter(idx_o, [hi, lo], oidx, mask=valid)
            plsc.store_scatter(idx_m, [hi, lo], midx, mask=valid)
            return off + jnp.sum(valid.astype(jnp.int32))
        off = lax.fori_loop(0, GROUPS, _scan, jnp.int32(0))

        def _append(k, off):
            sv = sentb[pl.ds(k * 16, 16)]
            lv = lane + k * 16
            valid = (lv < L) & (wid == 0)
            nid = plsc.load_gather(nids, [sv])
            oidx = XBASE + lv
            cs = plsc.cumsum(valid.astype(jnp.int32))
            pos = jnp.minimum(off + cs - 1, CAP - 1)
            hi = pos >> 5
            lo = pos & 31
            plsc.store_scatter(idx_s, [hi, lo], nid, mask=valid)
            plsc.store_scatter(idx_o, [hi, lo], oidx, mask=valid)
            plsc.store_scatter(idx_m, [hi, lo], zero16i, mask=valid)
            return off + jnp.sum(valid.astype(jnp.int32))
        off = lax.fori_loop(0, LP // 16, _append, off)

        nch = jnp.minimum((off + 31) >> 5, NCH)

        def _flush(j, _):
            pltpu.async_copy(emb_hbm.at[idx_s.at[j]], rows, sem).wait()
            pltpu.sync_copy(rows, sg.at[g].at[idx_o.at[j]], add=True)
            pltpu.sync_copy(ones_r, degg.at[g].at[idx_m.at[j]], add=True)
            return 0
        lax.fori_loop(0, nch, _flush, 0)

    plsc.subcore_barrier()

    idx = 0
    for g in range(B):
        for ch in range(SROWS // 16):
            @pl.when(ss == (idx % NS))
            def _():
                pltpu.sync_copy(sg.at[g, pl.ds(ch * 16, 16)],
                                s_out.at[cc, g, pl.ds(ch * 16, 16)])
            idx += 1
    for g in range(B):
        for ch in range(LP // 16):
            @pl.when(ss == ((g * (LP // 16) + ch) % NS))
            def _():
                pltpu.sync_copy(degg.at[g, pl.ds(ch * 16, 16)],
                                deg_out.at[cc, g, pl.ds(ch * 16, 16)])


@jax.jit
def _sc_stage(ei, et, nids, sent, emb):
    mesh = plsc.VectorSubcoreMesh(core_axis_name="c", subcore_axis_name="s")
    f = pl.kernel(
        _sc_body,
        out_type=[
            jax.ShapeDtypeStruct((NC, B, SROWS, D), jnp.float32),
            jax.ShapeDtypeStruct((NC, B, LP, 16), jnp.float32),
        ],
        mesh=mesh,
        compiler_params=pltpu.CompilerParams(needs_layout_passes=False),
        scratch_types=[
            pltpu.VMEM((N_NODES,), jnp.int32),
            pltpu.VMEM((N_NODES,), jnp.int32),
            pltpu.VMEM((EPT,), jnp.int32),
            pltpu.VMEM((EPT,), jnp.int32),
            pltpu.VMEM((EPT,), jnp.int32),
            pltpu.VMEM((LP,), jnp.int32),
            pltpu.VMEM((NCH, 32), jnp.int32),
            pltpu.VMEM((NCH, 32), jnp.int32),
            pltpu.VMEM((NCH, 32), jnp.int32),
            pltpu.VMEM((32, D), jnp.float32),
            pltpu.VMEM((32, 16), jnp.float32),
            pltpu.VMEM((16, D), jnp.float32),
            pltpu.VMEM((16, 16), jnp.float32),
            pltpu.VMEM_SHARED((B, SROWS, D), jnp.float32),
            pltpu.VMEM_SHARED((B, LP, 16), jnp.float32),
            pltpu.SemaphoreType.DMA,
        ],
    )
    return f(ei, et, nids, sent, emb)


def _tc_body(s_ref, deg_ref, sent_ref, basis_ref, comp_ref, root_ref, bias_ref,
             wq_ref, wk_ref, wv_ref, attw_ref, attb_ref, fcw_ref, fcb_ref,
             out_ref):
    s = s_ref[0] + s_ref[1]
    deg = jnp.sum(deg_ref[0] + deg_ref[1], axis=-1) * (1.0 / 16.0)

    w = jnp.einsum("rb,bde->rde", comp_ref[...], basis_ref[...])
    msum = jnp.zeros((B, L, D), jnp.float32)
    for r in range(R):
        msum = msum + jnp.einsum("gld,de->gle", s[:, 1 + r * L:1 + (r + 1) * L, :], w[r])

    sent = sent_ref[...][:, :L]
    p = (sent[:, :, None] == sent[:, None, :]).astype(jnp.float32)
    sum_s = jnp.einsum("glk,gkd->gld", p, msum)
    sum_deg = jnp.einsum("glk,gk->gl", p, deg[:, 1:1 + L])
    agg = sum_s / jnp.clip(sum_deg, 1.0, None)[..., None]

    xs = s[:, XBASE:XBASE + L, :]
    x = jnp.einsum("gld,de->gle", xs, root_ref[...]) + agg + bias_ref[...]

    def heads(wref):
        hh = jnp.einsum("gld,hde->glhe", x, wref[...])
        return hh.reshape(B, L, HEADS * H)
    q = heads(wq_ref)
    k = heads(wk_ref)
    v = heads(wv_ref)
    score = jnp.einsum("gle,gme->glm", q, k) * (1.0 / jnp.sqrt(float(HEADS * H)))
    score = jax.nn.softmax(score, axis=-1)
    hidden = jnp.einsum("glm,gme->gle", score, v)
    attn = jnp.einsum("gle,eo->glo", hidden, attw_ref[...]) + attb_ref[...]
    pooled = jnp.sum(hidden * attn, axis=1)
    logits = pooled @ fcw_ref[...] + fcb_ref[...]
    out_ref[...] = 1.0 / (1.0 + jnp.exp(-logits))


@jax.jit
def _tc_stage(s, deg, sent, basis, comp, root, bias, wq, wk, wv, attw, attb,
              fcw, fcb):
    return pl.pallas_call(
        _tc_body,
        out_shape=jax.ShapeDtypeStruct((B, 1), jnp.float32),
    )(s, deg, sent, basis, comp, root, bias, wq, wk, wv, attw, attb, fcw, fcb)


def kernel(sent_ids, edge_index, edge_type, node_ids, emb, basis, comp, root,
           bias, WQ, WK, WV, attW, attb, fcW, fcb):
    ei = jnp.pad(edge_index.astype(jnp.int32), ((0, 0), (0, 0), (0, PE - E)))
    et = jnp.pad(edge_type.astype(jnp.int32), ((0, 0), (0, PE - E)))
    sent = jnp.pad(sent_ids.astype(jnp.int32), ((0, 0), (0, LP - L)))
    nids = node_ids.astype(jnp.int32)
    s, deg = _sc_stage(ei, et, nids, sent, emb)
    return _tc_stage(s, deg, sent, basis, comp, root, bias, WQ, WK, WV,
                     attW, attb, fcW, fcb)
